# baseline (speedup 1.0000x reference)
import functools

import jax
import jax.numpy as jnp
from jax import lax
from jax.experimental import pallas as pl
from jax.experimental.pallas import tpu as pltpu
from jax.experimental.pallas import tpu_sc as plsc

N = 10000
E = 320000
WIDTH = 128
NUM_HEAD = 8
GSIZE = WIDTH // NUM_HEAD
BOND = 33
EPS = 1e-5

_PRE_BLK = 2000
_MSG_BLK = 1280
_WIN = 128
_SC_TILES = 16


def _pack_bf16_halves(h):
    lo = lax.bitcast_convert_type(h[:, :64].astype(jnp.bfloat16), jnp.uint16)
    hi = lax.bitcast_convert_type(h[:, 64:].astype(jnp.bfloat16), jnp.uint16)
    word = (hi.astype(jnp.uint32) << 16) | lo.astype(jnp.uint32)
    return lax.bitcast_convert_type(word, jnp.int32)


def _unpack_bf16_halves(w):
    lo = lax.bitcast_convert_type(w << 16, jnp.float32)
    hi = lax.bitcast_convert_type(w & jnp.int32(-65536), jnp.float32)
    return lo, hi


def _pre_body(x_ref, w0_ref, b0_ref, w1_ref, b1_ref, t_ref):
    x = x_ref[...].astype(jnp.bfloat16)
    h0 = jnp.dot(x, w0_ref[...], preferred_element_type=jnp.float32) + b0_ref[...]
    h1 = jnp.dot(x, w1_ref[...], preferred_element_type=jnp.float32) + b1_ref[...]
    t_ref[...] = jnp.concatenate(
        [_pack_bf16_halves(h0), _pack_bf16_halves(h1)], axis=1)


def _msg_body(gm_ref, attr_ref, tabg_ref, m_ref, wg_ref, wv_ref, msg_ref):
    bf = jnp.bfloat16
    w = gm_ref[...]
    lo0, hi0 = _unpack_bf16_halves(w[:, :WIDTH // 2])
    lo1, hi1 = _unpack_bf16_halves(w[:, WIDTH // 2:])
    g = jnp.concatenate([lo0 + lo1, hi0 + hi1], axis=1)
    gb = g.astype(bf)
    m = m_ref[...]
    mu = jnp.dot(gb, m, preferred_element_type=jnp.float32)
    msq = jnp.dot(gb * gb, m, preferred_element_type=jnp.float32)
    r = lax.rsqrt(jnp.maximum(msq - mu * mu, 0.0) + EPS)
    xc = (g - mu).astype(bf)
    tg = jnp.dot(xc, wg_ref[...], preferred_element_type=jnp.float32)
    tv = jnp.dot(xc, wv_ref[...], preferred_element_type=jnp.float32)

    attr = attr_ref[...]
    ab = attr.astype(bf)
    laneb = lax.broadcasted_iota(jnp.int32, (_MSG_BLK, WIDTH), 1).astype(bf)
    c = jnp.zeros((_MSG_BLK, WIDTH), bf)
    for k in range(4):
        c = c + jnp.where(laneb == ab[:, k][:, None],
                          jnp.ones((), bf), jnp.zeros((), bf))
    bg = jnp.dot(c, tabg_ref[...], preferred_element_type=jnp.float32)
    cnt = jnp.sum((attr != 0).astype(jnp.float32), axis=1, keepdims=True)
    rc = 1.0 / jnp.maximum(cnt, 1.0)

    gate = jnp.maximum(tg * r + bg * rc, 0.0)
    msg_ref[...] = gate * (tv * r)


def _post_body(a0_ref, a1_ref, wp_ref, bp_ref, deg_ref, dp_ref, xres_ref, out_ref):
    agg = a0_ref[...] + a1_ref[...]
    out = jnp.dot(agg.astype(jnp.bfloat16), wp_ref[...],
                  preferred_element_type=jnp.float32) + bp_ref[...]
    scale = jnp.exp(dp_ref[...] * jnp.log(deg_ref[...]))
    out_ref[...] = scale * out + xres_ref[...]


_MESH = plsc.VectorSubcoreMesh(core_axis_name="core", subcore_axis_name="subcore")


_NWIN = E // _WIN
_WPW = -(-_NWIN // 32)
_REM = _NWIN - (_NWIN // 32) * 32
_BASE_W = _NWIN // 32


def _gather(t, src_pad, dst_pad):
    half = WIDTH // 2

    @functools.partial(
        pl.kernel,
        out_type=jax.ShapeDtypeStruct((E, WIDTH), jnp.int32),
        mesh=_MESH,
        scratch_types=[pltpu.VMEM((_WPW * _WIN,), jnp.int32),
                       pltpu.VMEM((_WPW * _WIN,), jnp.int32),
                       pltpu.VMEM((2, _WIN, WIDTH), jnp.int32),
                       pltpu.VMEM((2, _WIN, WIDTH), jnp.int32),
                       pltpu.SemaphoreType.DMA,
                       pltpu.SemaphoreType.DMA,
                       pltpu.SemaphoreType.DMA])
    def k(t_hbm, src_hbm, dst_hbm, gm_hbm, i0v, i1v, s0, s1,
          sem0, sem1, wsem):
        cid = lax.axis_index("core")
        sid = lax.axis_index("subcore")
        w = sid * 2 + cid
        nwin = _BASE_W + jnp.where(w < _REM, 1, 0)
        base_win = w * _BASE_W + jnp.minimum(w, _REM)
        base_e = base_win * _WIN
        pltpu.sync_copy(src_hbm.at[0, pl.ds(base_e, _WPW * _WIN)], i0v)
        pltpu.sync_copy(dst_hbm.at[0, pl.ds(base_e, _WPW * _WIN)], i1v)

        def gather_cp(j, slot):
            cp0 = pltpu.make_async_copy(
                t_hbm.at[i0v.at[pl.ds(j * _WIN, _WIN)]], s0.at[slot], sem0)
            cp1 = pltpu.make_async_copy(
                t_hbm.at[i1v.at[pl.ds(j * _WIN, _WIN)]], s1.at[slot], sem1)
            return cp0, cp1

        def write_cp(j, slot):
            return pltpu.make_async_copy(
                s0.at[slot], gm_hbm.at[pl.ds(base_e + j * _WIN, _WIN)], wsem)

        def fire(j, slot):
            cp0, cp1 = gather_cp(j, slot)
            cp0.start()
            cp1.start()

        @pl.when(nwin > 0)
        def _():
            fire(0, 0)

        @pl.when(nwin > 1)
        def _():
            fire(1, 1)

        @pl.loop(0, _WPW)
        def _(j):
            slot = lax.rem(j, 2)

            @pl.when(j < nwin)
            def _():
                cp0, cp1 = gather_cp(j, slot)
                cp0.wait()
                cp1.wait()

                @pl.loop(0, _WIN)
                def _(row):
                    for cc in range(half // 16):
                        col = half + cc * 16
                        s0[slot, row, pl.ds(col, 16)] = (
                            s1[slot, row, pl.ds(col, 16)])

                cw = write_cp(j, slot)
                cw.start()
                cw.wait()

                @pl.when(j + 2 < nwin)
                def _():
                    fire(j + 2, slot)

    return k(t, src_pad, dst_pad)


_NPAD = 10240


def _scatter(msg, dst, zeros):
    rows = _NPAD // _SC_TILES

    @functools.partial(
        pl.kernel,
        out_type=jax.ShapeDtypeStruct((2, _NPAD, WIDTH), jnp.float32),
        mesh=_MESH,
        scratch_types=[pltpu.VMEM_SHARED((_NPAD, WIDTH), jnp.float32)])
    def k(msg_hbm, dst_hbm, z_hbm, out_hbm, acc):
        cid = lax.axis_index("core")
        sid = lax.axis_index("subcore")
        pltpu.sync_copy(z_hbm.at[pl.ds(sid * rows, rows)],
                        acc.at[pl.ds(sid * rows, rows)])
        plsc.subcore_barrier()

        def body(m_v, i_v):
            pltpu.sync_copy(m_v, acc.at[i_v.at[0]], add=True)

        pltpu.emit_pipeline(
            body,
            grid=(E // _WIN,),
            in_specs=[pl.BlockSpec((_WIN, WIDTH), lambda i: (i, 0)),
                      pl.BlockSpec((1, _WIN), lambda i: (0, i))],
            out_specs=[],
            core_axis_name=("core", "subcore"),
            dimension_semantics=(pltpu.PARALLEL,),
        )(msg_hbm, dst_hbm)
        plsc.subcore_barrier()
        pltpu.sync_copy(acc.at[pl.ds(sid * rows, rows)],
                        out_hbm.at[cid, pl.ds(sid * rows, rows)])

    return k(msg, dst, zeros)


def kernel(x, x_res, edge_index, edge_attr, node_deg, W_pre0, b_pre0, W_pre1,
           b_pre1, emb_table, W_gate, W_value, W_post, b_post, degree_param):
    f32 = jnp.float32
    src = edge_index[0].reshape(1, E)
    dst = edge_index[1].reshape(1, E)

    wp_specs = [
        pl.BlockSpec((_PRE_BLK, WIDTH), lambda i: (i, 0)),
        pl.BlockSpec((WIDTH, WIDTH), lambda i: (0, 0)),
        pl.BlockSpec((1, WIDTH), lambda i: (0, 0)),
        pl.BlockSpec((WIDTH, WIDTH), lambda i: (0, 0)),
        pl.BlockSpec((1, WIDTH), lambda i: (0, 0)),
    ]
    h01 = pl.pallas_call(
        _pre_body,
        grid=(N // _PRE_BLK,),
        in_specs=wp_specs,
        out_specs=pl.BlockSpec((_PRE_BLK, WIDTH), lambda i: (i, 0)),
        out_shape=jax.ShapeDtypeStruct((N, WIDTH), jnp.int32),
    )(x, W_pre0.T, b_pre0.reshape(1, WIDTH), W_pre1.T, b_pre1.reshape(1, WIDTH))

    pad = jnp.zeros((1, _WIN), jnp.int32)
    gm = _gather(h01, jnp.concatenate([src, pad], axis=1),
                 jnp.concatenate([dst, pad], axis=1))

    eye8 = jnp.eye(NUM_HEAD, dtype=f32)
    wg_full = jnp.einsum(
        'goc,gh->gcho', W_gate.reshape(NUM_HEAD, GSIZE, GSIZE), eye8
    ).reshape(WIDTH, WIDTH)
    wv_full = jnp.einsum(
        'goc,gh->gcho', W_value.reshape(NUM_HEAD, GSIZE, GSIZE), eye8
    ).reshape(WIDTH, WIDTH)
    m_full = (jnp.einsum('gh,co->gcho', eye8, jnp.ones((GSIZE, GSIZE), f32))
              / GSIZE).reshape(WIDTH, WIDTH)
    tab_pad = jnp.zeros((WIDTH, WIDTH), f32).at[:BOND].set(emb_table).at[0].set(0.0)
    tabg = tab_pad @ wg_full
    wg_bf = wg_full.astype(jnp.bfloat16)
    wv_bf = wv_full.astype(jnp.bfloat16)
    m_bf = m_full.astype(jnp.bfloat16)
    tabg_bf = tabg.astype(jnp.bfloat16)

    msg = pl.pallas_call(
        _msg_body,
        grid=(E // _MSG_BLK,),
        in_specs=[
            pl.BlockSpec((_MSG_BLK, WIDTH), lambda i: (i, 0)),
            pl.BlockSpec((_MSG_BLK, 4), lambda i: (i, 0)),
            pl.BlockSpec((WIDTH, WIDTH), lambda i: (0, 0)),
            pl.BlockSpec((WIDTH, WIDTH), lambda i: (0, 0)),
            pl.BlockSpec((WIDTH, WIDTH), lambda i: (0, 0)),
            pl.BlockSpec((WIDTH, WIDTH), lambda i: (0, 0)),
        ],
        out_specs=pl.BlockSpec((_MSG_BLK, WIDTH), lambda i: (i, 0)),
        out_shape=jax.ShapeDtypeStruct((E, WIDTH), f32),
    )(gm, edge_attr, tabg_bf, m_bf, wg_bf, wv_bf)

    aggs = _scatter(msg, dst, jnp.zeros((_NPAD, WIDTH), f32))

    out = pl.pallas_call(
        _post_body,
        grid=(N // _PRE_BLK,),
        in_specs=[
            pl.BlockSpec((_PRE_BLK, WIDTH), lambda i: (i, 0)),
            pl.BlockSpec((_PRE_BLK, WIDTH), lambda i: (i, 0)),
            pl.BlockSpec((WIDTH, WIDTH), lambda i: (0, 0)),
            pl.BlockSpec((1, WIDTH), lambda i: (0, 0)),
            pl.BlockSpec((_PRE_BLK, 1), lambda i: (i, 0)),
            pl.BlockSpec((1, WIDTH), lambda i: (0, 0)),
            pl.BlockSpec((_PRE_BLK, WIDTH), lambda i: (i, 0)),
        ],
        out_specs=pl.BlockSpec((_PRE_BLK, WIDTH), lambda i: (i, 0)),
        out_shape=jax.ShapeDtypeStruct((N, WIDTH), f32),
    )(aggs[0], aggs[1], W_post.T, b_post.reshape(1, WIDTH),
      node_deg.reshape(N, 1), degree_param.reshape(1, WIDTH), x_res)

    return out


# speedup vs baseline: 1.0985x; 1.0985x over previous
import functools

import jax
import jax.numpy as jnp
from jax import lax
from jax.experimental import pallas as pl
from jax.experimental.pallas import tpu as pltpu
from jax.experimental.pallas import tpu_sc as plsc

N = 10000
E = 320000
WIDTH = 128
NUM_HEAD = 8
GSIZE = WIDTH // NUM_HEAD
BOND = 33
EPS = 1e-5

_PRE_BLK = 2000
_MSG_BLK = 1280
_WIN = 128
_SC_TILES = 16


def _pack_bf16_halves(h):
    lo = lax.bitcast_convert_type(h[:, :64].astype(jnp.bfloat16), jnp.uint16)
    hi = lax.bitcast_convert_type(h[:, 64:].astype(jnp.bfloat16), jnp.uint16)
    word = (hi.astype(jnp.uint32) << 16) | lo.astype(jnp.uint32)
    return lax.bitcast_convert_type(word, jnp.int32)


def _unpack_bf16_halves(w):
    lo = lax.bitcast_convert_type(w << 16, jnp.float32)
    hi = lax.bitcast_convert_type(w & jnp.int32(-65536), jnp.float32)
    return lo, hi


def _pre_body(x_ref, w0_ref, b0_ref, w1_ref, b1_ref, t_ref):
    x = x_ref[...].astype(jnp.bfloat16)
    h0 = jnp.dot(x, w0_ref[...], preferred_element_type=jnp.float32) + b0_ref[...]
    h1 = jnp.dot(x, w1_ref[...], preferred_element_type=jnp.float32) + b1_ref[...]
    t_ref[...] = jnp.concatenate(
        [_pack_bf16_halves(h0), _pack_bf16_halves(h1)], axis=1)


def _msg_body(gm_ref, attr_ref, tabg_ref, m_ref, wg_ref, wv_ref, msg_ref):
    bf = jnp.bfloat16
    w = gm_ref[...]
    lo0, hi0 = _unpack_bf16_halves(w[:, :WIDTH // 2])
    lo1, hi1 = _unpack_bf16_halves(w[:, WIDTH // 2:])
    g = jnp.concatenate([lo0 + lo1, hi0 + hi1], axis=1)
    gb = g.astype(bf)
    m = m_ref[...]
    mu = jnp.dot(gb, m, preferred_element_type=jnp.float32)
    msq = jnp.dot(gb * gb, m, preferred_element_type=jnp.float32)
    r = lax.rsqrt(jnp.maximum(msq - mu * mu, 0.0) + EPS)
    xc = (g - mu).astype(bf)
    tg = jnp.dot(xc, wg_ref[...], preferred_element_type=jnp.float32)
    tv = jnp.dot(xc, wv_ref[...], preferred_element_type=jnp.float32)

    attr = attr_ref[...]
    ab = attr.astype(bf)
    laneb = lax.broadcasted_iota(jnp.int32, (_MSG_BLK, WIDTH), 1).astype(bf)
    c = jnp.zeros((_MSG_BLK, WIDTH), bf)
    for k in range(4):
        c = c + jnp.where(laneb == ab[:, k][:, None],
                          jnp.ones((), bf), jnp.zeros((), bf))
    bg = jnp.dot(c, tabg_ref[...], preferred_element_type=jnp.float32)
    cnt = jnp.sum((attr != 0).astype(jnp.float32), axis=1, keepdims=True)
    rc = 1.0 / jnp.maximum(cnt, 1.0)

    gate = jnp.maximum(tg * r + bg * rc, 0.0)
    msg_ref[...] = gate * (tv * r)


def _post_body(*refs):
    (a_refs, (wp_ref, bp_ref, deg_ref, dp_ref, xres_ref, out_ref)) = (
        refs[:-6], refs[-6:])
    agg = a_refs[0][...]
    for a in a_refs[1:]:
        agg = agg + a[...]
    out = jnp.dot(agg.astype(jnp.bfloat16), wp_ref[...],
                  preferred_element_type=jnp.float32) + bp_ref[...]
    scale = jnp.exp(dp_ref[...] * jnp.log(deg_ref[...]))
    out_ref[...] = scale * out + xres_ref[...]


_MESH = plsc.VectorSubcoreMesh(core_axis_name="core", subcore_axis_name="subcore")


_NCH = 2
_EC = E // _NCH
_NWINC = _EC // _WIN
_WPW = -(-_NWINC // 32)
_REM = _NWINC - (_NWINC // 32) * 32
_BASE_W = _NWINC // 32


def _gather(t, src_pad, dst_pad):
    half = WIDTH // 2

    @functools.partial(
        pl.kernel,
        out_type=jax.ShapeDtypeStruct((_EC, WIDTH), jnp.int32),
        mesh=_MESH,
        scratch_types=[pltpu.VMEM((_WPW * _WIN,), jnp.int32),
                       pltpu.VMEM((_WPW * _WIN,), jnp.int32),
                       pltpu.VMEM((2, _WIN, WIDTH), jnp.int32),
                       pltpu.VMEM((2, _WIN, WIDTH), jnp.int32),
                       pltpu.SemaphoreType.DMA,
                       pltpu.SemaphoreType.DMA,
                       pltpu.SemaphoreType.DMA])
    def k(t_hbm, src_hbm, dst_hbm, gm_hbm, i0v, i1v, s0, s1,
          sem0, sem1, wsem):
        cid = lax.axis_index("core")
        sid = lax.axis_index("subcore")
        w = sid * 2 + cid
        nwin = _BASE_W + jnp.where(w < _REM, 1, 0)
        base_win = w * _BASE_W + jnp.minimum(w, _REM)
        base_e = base_win * _WIN
        pltpu.sync_copy(src_hbm.at[0, pl.ds(base_e, _WPW * _WIN)], i0v)
        pltpu.sync_copy(dst_hbm.at[0, pl.ds(base_e, _WPW * _WIN)], i1v)

        def gather_cp(j, slot):
            cp0 = pltpu.make_async_copy(
                t_hbm.at[i0v.at[pl.ds(j * _WIN, _WIN)]], s0.at[slot], sem0)
            cp1 = pltpu.make_async_copy(
                t_hbm.at[i1v.at[pl.ds(j * _WIN, _WIN)]], s1.at[slot], sem1)
            return cp0, cp1

        def write_cp(j, slot):
            return pltpu.make_async_copy(
                s0.at[slot], gm_hbm.at[pl.ds(base_e + j * _WIN, _WIN)], wsem)

        def fire(j, slot):
            cp0, cp1 = gather_cp(j, slot)
            cp0.start()
            cp1.start()

        @pl.when(nwin > 0)
        def _():
            fire(0, 0)

        @pl.when(nwin > 1)
        def _():
            fire(1, 1)

        @pl.loop(0, _WPW)
        def _(j):
            slot = lax.rem(j, 2)

            @pl.when(j < nwin)
            def _():
                cp0, cp1 = gather_cp(j, slot)
                cp0.wait()
                cp1.wait()

                @pl.loop(0, _WIN)
                def _(row):
                    for cc in range(half // 16):
                        col = half + cc * 16
                        s0[slot, row, pl.ds(col, 16)] = (
                            s1[slot, row, pl.ds(col, 16)])

                cw = write_cp(j, slot)
                cw.start()
                cw.wait()

                @pl.when(j + 2 < nwin)
                def _():
                    fire(j + 2, slot)

    return k(t, src_pad, dst_pad)


_NPAD = 10240


def _scatter(msg, dst, zeros):
    rows = _NPAD // _SC_TILES

    @functools.partial(
        pl.kernel,
        out_type=jax.ShapeDtypeStruct((2, _NPAD, WIDTH), jnp.float32),
        mesh=_MESH,
        scratch_types=[pltpu.VMEM_SHARED((_NPAD, WIDTH), jnp.float32)])
    def k(msg_hbm, dst_hbm, z_hbm, out_hbm, acc):
        cid = lax.axis_index("core")
        sid = lax.axis_index("subcore")
        pltpu.sync_copy(z_hbm.at[pl.ds(sid * rows, rows)],
                        acc.at[pl.ds(sid * rows, rows)])
        plsc.subcore_barrier()

        def body(m_v, i_v):
            pltpu.sync_copy(m_v, acc.at[i_v.at[0]], add=True)

        pltpu.emit_pipeline(
            body,
            grid=(_EC // _WIN,),
            in_specs=[pl.BlockSpec((_WIN, WIDTH), lambda i: (i, 0)),
                      pl.BlockSpec((1, _WIN), lambda i: (0, i))],
            out_specs=[],
            core_axis_name=("core", "subcore"),
            dimension_semantics=(pltpu.PARALLEL,),
        )(msg_hbm, dst_hbm)
        plsc.subcore_barrier()
        pltpu.sync_copy(acc.at[pl.ds(sid * rows, rows)],
                        out_hbm.at[cid, pl.ds(sid * rows, rows)])

    return k(msg, dst, zeros)


def kernel(x, x_res, edge_index, edge_attr, node_deg, W_pre0, b_pre0, W_pre1,
           b_pre1, emb_table, W_gate, W_value, W_post, b_post, degree_param):
    f32 = jnp.float32
    src = edge_index[0].reshape(1, E)
    dst = edge_index[1].reshape(1, E)

    wp_specs = [
        pl.BlockSpec((_PRE_BLK, WIDTH), lambda i: (i, 0)),
        pl.BlockSpec((WIDTH, WIDTH), lambda i: (0, 0)),
        pl.BlockSpec((1, WIDTH), lambda i: (0, 0)),
        pl.BlockSpec((WIDTH, WIDTH), lambda i: (0, 0)),
        pl.BlockSpec((1, WIDTH), lambda i: (0, 0)),
    ]
    h01 = pl.pallas_call(
        _pre_body,
        grid=(N // _PRE_BLK,),
        in_specs=wp_specs,
        out_specs=pl.BlockSpec((_PRE_BLK, WIDTH), lambda i: (i, 0)),
        out_shape=jax.ShapeDtypeStruct((N, WIDTH), jnp.int32),
    )(x, W_pre0.T, b_pre0.reshape(1, WIDTH), W_pre1.T, b_pre1.reshape(1, WIDTH))

    eye8 = jnp.eye(NUM_HEAD, dtype=f32)
    wg_full = jnp.einsum(
        'goc,gh->gcho', W_gate.reshape(NUM_HEAD, GSIZE, GSIZE), eye8
    ).reshape(WIDTH, WIDTH)
    wv_full = jnp.einsum(
        'goc,gh->gcho', W_value.reshape(NUM_HEAD, GSIZE, GSIZE), eye8
    ).reshape(WIDTH, WIDTH)
    m_full = (jnp.einsum('gh,co->gcho', eye8, jnp.ones((GSIZE, GSIZE), f32))
              / GSIZE).reshape(WIDTH, WIDTH)
    tab_pad = jnp.zeros((WIDTH, WIDTH), f32).at[:BOND].set(emb_table).at[0].set(0.0)
    tabg = tab_pad @ wg_full
    wg_bf = wg_full.astype(jnp.bfloat16)
    wv_bf = wv_full.astype(jnp.bfloat16)
    m_bf = m_full.astype(jnp.bfloat16)
    tabg_bf = tabg.astype(jnp.bfloat16)

    pad = jnp.zeros((1, _WIN), jnp.int32)
    src_pad = jnp.concatenate([src, pad], axis=1)
    dst_pad = jnp.concatenate([dst, pad], axis=1)
    zeros = jnp.zeros((_NPAD, WIDTH), f32)
    partials = []
    for ch in range(_NCH):
        e0 = ch * _EC
        gm = _gather(h01,
                     lax.slice(src_pad, (0, e0), (1, e0 + _EC + _WIN)),
                     lax.slice(dst_pad, (0, e0), (1, e0 + _EC + _WIN)))
        msg = pl.pallas_call(
            _msg_body,
            grid=(_EC // _MSG_BLK,),
            in_specs=[
                pl.BlockSpec((_MSG_BLK, WIDTH), lambda i: (i, 0)),
                pl.BlockSpec((_MSG_BLK, 4), lambda i: (i, 0)),
                pl.BlockSpec((WIDTH, WIDTH), lambda i: (0, 0)),
                pl.BlockSpec((WIDTH, WIDTH), lambda i: (0, 0)),
                pl.BlockSpec((WIDTH, WIDTH), lambda i: (0, 0)),
                pl.BlockSpec((WIDTH, WIDTH), lambda i: (0, 0)),
            ],
            out_specs=pl.BlockSpec((_MSG_BLK, WIDTH), lambda i: (i, 0)),
            out_shape=jax.ShapeDtypeStruct((_EC, WIDTH), f32),
        )(gm, lax.slice(edge_attr, (e0, 0), (e0 + _EC, 4)),
          tabg_bf, m_bf, wg_bf, wv_bf)
        aggs = _scatter(msg, lax.slice(dst, (0, e0), (1, e0 + _EC)), zeros)
        partials.append(aggs[0])
        partials.append(aggs[1])

    nagg = 2 * _NCH
    out = pl.pallas_call(
        _post_body,
        grid=(N // _PRE_BLK,),
        in_specs=(
            [pl.BlockSpec((_PRE_BLK, WIDTH), lambda i: (i, 0))] * nagg
            + [
                pl.BlockSpec((WIDTH, WIDTH), lambda i: (0, 0)),
                pl.BlockSpec((1, WIDTH), lambda i: (0, 0)),
                pl.BlockSpec((_PRE_BLK, 1), lambda i: (i, 0)),
                pl.BlockSpec((1, WIDTH), lambda i: (0, 0)),
                pl.BlockSpec((_PRE_BLK, WIDTH), lambda i: (i, 0)),
            ]
        ),
        out_specs=pl.BlockSpec((_PRE_BLK, WIDTH), lambda i: (i, 0)),
        out_shape=jax.ShapeDtypeStruct((N, WIDTH), f32),
    )(*partials, W_post.T, b_post.reshape(1, WIDTH),
      node_deg.reshape(N, 1), degree_param.reshape(1, WIDTH), x_res)

    return out


# speedup vs baseline: 1.2095x; 1.1010x over previous
import dataclasses
import functools

import jax
import jax.numpy as jnp
from jax import lax
from jax.experimental import pallas as pl
from jax.experimental.pallas import tpu as pltpu
from jax.experimental.pallas import tpu_sc as plsc

N = 10000
E = 320000
WIDTH = 128
NUM_HEAD = 8
GSIZE = WIDTH // NUM_HEAD
BOND = 33
EPS = 1e-5

_PRE_BLK = 2000
_MSG_BLK = 1280
_WIN = 128
_SC_TILES = 16


def _pack_bf16_halves(h):
    lo = lax.bitcast_convert_type(h[:, :64].astype(jnp.bfloat16), jnp.uint16)
    hi = lax.bitcast_convert_type(h[:, 64:].astype(jnp.bfloat16), jnp.uint16)
    word = (hi.astype(jnp.uint32) << 16) | lo.astype(jnp.uint32)
    return lax.bitcast_convert_type(word, jnp.int32)


def _unpack_bf16_halves(w):
    lo = lax.bitcast_convert_type(w << 16, jnp.float32)
    hi = lax.bitcast_convert_type(w & jnp.int32(-65536), jnp.float32)
    return lo, hi


def _pre_body(x_ref, w0_ref, b0_ref, w1_ref, b1_ref, t_ref):
    x = x_ref[...].astype(jnp.bfloat16)
    h0 = jnp.dot(x, w0_ref[...], preferred_element_type=jnp.float32) + b0_ref[...]
    h1 = jnp.dot(x, w1_ref[...], preferred_element_type=jnp.float32) + b1_ref[...]
    t_ref[...] = jnp.concatenate(
        [_pack_bf16_halves(h0), _pack_bf16_halves(h1)], axis=1)


def _msg_body(gm_ref, attr_ref, tabg_ref, m_ref, wg_ref, wv_ref, msg_ref):
    bf = jnp.bfloat16
    gb = gm_ref[...]
    g = gb.astype(jnp.float32)
    m = m_ref[...]
    mu = jnp.dot(gb, m, preferred_element_type=jnp.float32)
    msq = jnp.dot(gb * gb, m, preferred_element_type=jnp.float32)
    r = lax.rsqrt(jnp.maximum(msq - mu * mu, 0.0) + EPS)
    xc = (g - mu).astype(bf)
    tg = jnp.dot(xc, wg_ref[...], preferred_element_type=jnp.float32)
    tv = jnp.dot(xc, wv_ref[...], preferred_element_type=jnp.float32)

    attr = attr_ref[...]
    ab = attr.astype(bf)
    laneb = lax.broadcasted_iota(jnp.int32, (_MSG_BLK, WIDTH), 1).astype(bf)
    c = jnp.zeros((_MSG_BLK, WIDTH), bf)
    for k in range(4):
        c = c + jnp.where(laneb == ab[:, k][:, None],
                          jnp.ones((), bf), jnp.zeros((), bf))
    bg = jnp.dot(c, tabg_ref[...], preferred_element_type=jnp.float32)
    cnt = jnp.sum((attr != 0).astype(jnp.float32), axis=1, keepdims=True)
    rc = 1.0 / jnp.maximum(cnt, 1.0)

    gate = jnp.maximum(tg * r + bg * rc, 0.0)
    msg_ref[...] = gate * (tv * r)


def _post_body(*refs):
    (a_refs, (wp_ref, bp_ref, deg_ref, dp_ref, xres_ref, out_ref)) = (
        refs[:-6], refs[-6:])
    agg = a_refs[0][...]
    for a in a_refs[1:]:
        agg = agg + a[...]
    out = jnp.dot(agg.astype(jnp.bfloat16), wp_ref[...],
                  preferred_element_type=jnp.float32) + bp_ref[...]
    scale = jnp.exp(dp_ref[...] * jnp.log(deg_ref[...]))
    out_ref[...] = scale * out + xres_ref[...]


_MESH = plsc.VectorSubcoreMesh(core_axis_name="core", subcore_axis_name="subcore")

_SC_PARAMS = pltpu.CompilerParams()
if "needs_layout_passes" in pltpu.CompilerParams.__dataclass_fields__:
    _SC_PARAMS = dataclasses.replace(_SC_PARAMS, needs_layout_passes=False)


_NCH = 2
_EC = E // _NCH
_NWINC = _EC // _WIN
_WPW = -(-_NWINC // 32)
_REM = _NWINC - (_NWINC // 32) * 32
_BASE_W = _NWINC // 32


def _gather(t, src_pad, dst_pad):
    half = WIDTH // 2

    @functools.partial(
        pl.kernel,
        out_type=jax.ShapeDtypeStruct((_EC, WIDTH), jnp.bfloat16),
        mesh=_MESH,
        compiler_params=_SC_PARAMS,
        scratch_types=[pltpu.VMEM((_WPW * _WIN,), jnp.int32),
                       pltpu.VMEM((_WPW * _WIN,), jnp.int32),
                       pltpu.VMEM((2, _WIN, WIDTH), jnp.int32),
                       pltpu.VMEM((2, _WIN, WIDTH), jnp.int32),
                       pltpu.VMEM((2, _WIN, WIDTH), jnp.bfloat16),
                       pltpu.SemaphoreType.DMA,
                       pltpu.SemaphoreType.DMA,
                       pltpu.SemaphoreType.DMA])
    def k(t_hbm, src_hbm, dst_hbm, gm_hbm, i0v, i1v, s0, s1, sout,
          sem0, sem1, wsem):
        cid = lax.axis_index("core")
        sid = lax.axis_index("subcore")
        w = sid * 2 + cid
        nwin = _BASE_W + jnp.where(w < _REM, 1, 0)
        base_win = w * _BASE_W + jnp.minimum(w, _REM)
        base_e = base_win * _WIN
        pltpu.sync_copy(src_hbm.at[0, pl.ds(base_e, _WPW * _WIN)], i0v)
        pltpu.sync_copy(dst_hbm.at[0, pl.ds(base_e, _WPW * _WIN)], i1v)

        def gather_cp(j, slot):
            cp0 = pltpu.make_async_copy(
                t_hbm.at[i0v.at[pl.ds(j * _WIN, _WIN)]], s0.at[slot], sem0)
            cp1 = pltpu.make_async_copy(
                t_hbm.at[i1v.at[pl.ds(j * _WIN, _WIN)]], s1.at[slot], sem1)
            return cp0, cp1

        def write_cp(j, slot):
            return pltpu.make_async_copy(
                sout.at[slot], gm_hbm.at[pl.ds(base_e + j * _WIN, _WIN)], wsem)

        def fire(j, slot):
            cp0, cp1 = gather_cp(j, slot)
            cp0.start()
            cp1.start()

        @pl.when(nwin > 0)
        def _():
            fire(0, 0)

        @pl.when(nwin > 1)
        def _():
            fire(1, 1)

        @pl.loop(0, _WPW)
        def _(j):
            slot = lax.rem(j, 2)

            @pl.when(j < nwin)
            def _():
                cp0, cp1 = gather_cp(j, slot)
                cp0.wait()
                cp1.wait()

                @pl.loop(0, _WIN)
                def _(row):
                    for cc in range(half // 16):
                        a = plsc.bitcast(s0[slot, row, pl.ds(cc * 16, 16)],
                                         jnp.bfloat16)
                        b = plsc.bitcast(
                            s1[slot, row, pl.ds(half + cc * 16, 16)],
                            jnp.bfloat16)
                        sout[slot, row, pl.ds(cc * 32, 32)] = a + b

                cw = write_cp(j, slot)
                cw.start()
                cw.wait()

                @pl.when(j + 2 < nwin)
                def _():
                    fire(j + 2, slot)

    return k(t, src_pad, dst_pad)


_NPAD = 10240


def _scatter(msg, dst, zeros):
    rows = _NPAD // _SC_TILES

    @functools.partial(
        pl.kernel,
        out_type=jax.ShapeDtypeStruct((2, _NPAD, WIDTH), jnp.float32),
        mesh=_MESH,
        scratch_types=[pltpu.VMEM_SHARED((_NPAD, WIDTH), jnp.float32)])
    def k(msg_hbm, dst_hbm, z_hbm, out_hbm, acc):
        cid = lax.axis_index("core")
        sid = lax.axis_index("subcore")
        pltpu.sync_copy(z_hbm.at[pl.ds(sid * rows, rows)],
                        acc.at[pl.ds(sid * rows, rows)])
        plsc.subcore_barrier()

        def body(m_v, i_v):
            pltpu.sync_copy(m_v, acc.at[i_v.at[0]], add=True)

        pltpu.emit_pipeline(
            body,
            grid=(_EC // _WIN,),
            in_specs=[pl.BlockSpec((_WIN, WIDTH), lambda i: (i, 0)),
                      pl.BlockSpec((1, _WIN), lambda i: (0, i))],
            out_specs=[],
            core_axis_name=("core", "subcore"),
            dimension_semantics=(pltpu.PARALLEL,),
        )(msg_hbm, dst_hbm)
        plsc.subcore_barrier()
        pltpu.sync_copy(acc.at[pl.ds(sid * rows, rows)],
                        out_hbm.at[cid, pl.ds(sid * rows, rows)])

    return k(msg, dst, zeros)


def kernel(x, x_res, edge_index, edge_attr, node_deg, W_pre0, b_pre0, W_pre1,
           b_pre1, emb_table, W_gate, W_value, W_post, b_post, degree_param):
    f32 = jnp.float32
    src = edge_index[0].reshape(1, E)
    dst = edge_index[1].reshape(1, E)

    wp_specs = [
        pl.BlockSpec((_PRE_BLK, WIDTH), lambda i: (i, 0)),
        pl.BlockSpec((WIDTH, WIDTH), lambda i: (0, 0)),
        pl.BlockSpec((1, WIDTH), lambda i: (0, 0)),
        pl.BlockSpec((WIDTH, WIDTH), lambda i: (0, 0)),
        pl.BlockSpec((1, WIDTH), lambda i: (0, 0)),
    ]
    h01 = pl.pallas_call(
        _pre_body,
        grid=(N // _PRE_BLK,),
        in_specs=wp_specs,
        out_specs=pl.BlockSpec((_PRE_BLK, WIDTH), lambda i: (i, 0)),
        out_shape=jax.ShapeDtypeStruct((N, WIDTH), jnp.int32),
    )(x, W_pre0.T, b_pre0.reshape(1, WIDTH), W_pre1.T, b_pre1.reshape(1, WIDTH))

    eye8 = jnp.eye(NUM_HEAD, dtype=f32)
    wg_full = jnp.einsum(
        'goc,gh->gcho', W_gate.reshape(NUM_HEAD, GSIZE, GSIZE), eye8
    ).reshape(WIDTH, WIDTH)
    wv_full = jnp.einsum(
        'goc,gh->gcho', W_value.reshape(NUM_HEAD, GSIZE, GSIZE), eye8
    ).reshape(WIDTH, WIDTH)
    m_full = (jnp.einsum('gh,co->gcho', eye8, jnp.ones((GSIZE, GSIZE), f32))
              / GSIZE).reshape(WIDTH, WIDTH)
    tab_pad = jnp.zeros((WIDTH, WIDTH), f32).at[:BOND].set(emb_table).at[0].set(0.0)
    tabg = tab_pad @ wg_full
    cc_ = jnp.arange(WIDTH) // 32
    k_ = (jnp.arange(WIDTH) % 32) // 2
    perm = 16 * cc_ + k_ + 64 * (jnp.arange(WIDTH) % 2)
    wg_bf = wg_full[perm][:, perm].astype(jnp.bfloat16)
    wv_bf = wv_full[perm][:, perm].astype(jnp.bfloat16)
    m_bf = m_full[perm][:, perm].astype(jnp.bfloat16)
    tabg_bf = tabg[:, perm].astype(jnp.bfloat16)
    wp_perm = W_post.T[perm, :]

    pad = jnp.zeros((1, _WIN), jnp.int32)
    src_pad = jnp.concatenate([src, pad], axis=1)
    dst_pad = jnp.concatenate([dst, pad], axis=1)
    zeros = jnp.zeros((_NPAD, WIDTH), f32)
    partials = []
    for ch in range(_NCH):
        e0 = ch * _EC
        gm = _gather(h01,
                     lax.slice(src_pad, (0, e0), (1, e0 + _EC + _WIN)),
                     lax.slice(dst_pad, (0, e0), (1, e0 + _EC + _WIN)))
        msg = pl.pallas_call(
            _msg_body,
            grid=(_EC // _MSG_BLK,),
            in_specs=[
                pl.BlockSpec((_MSG_BLK, WIDTH), lambda i: (i, 0)),
                pl.BlockSpec((_MSG_BLK, 4), lambda i: (i, 0)),
                pl.BlockSpec((WIDTH, WIDTH), lambda i: (0, 0)),
                pl.BlockSpec((WIDTH, WIDTH), lambda i: (0, 0)),
                pl.BlockSpec((WIDTH, WIDTH), lambda i: (0, 0)),
                pl.BlockSpec((WIDTH, WIDTH), lambda i: (0, 0)),
            ],
            out_specs=pl.BlockSpec((_MSG_BLK, WIDTH), lambda i: (i, 0)),
            out_shape=jax.ShapeDtypeStruct((_EC, WIDTH), f32),
        )(gm, lax.slice(edge_attr, (e0, 0), (e0 + _EC, 4)),
          tabg_bf, m_bf, wg_bf, wv_bf)
        aggs = _scatter(msg, lax.slice(dst, (0, e0), (1, e0 + _EC)), zeros)
        partials.append(aggs[0])
        partials.append(aggs[1])

    nagg = 2 * _NCH
    out = pl.pallas_call(
        _post_body,
        grid=(N // _PRE_BLK,),
        in_specs=(
            [pl.BlockSpec((_PRE_BLK, WIDTH), lambda i: (i, 0))] * nagg
            + [
                pl.BlockSpec((WIDTH, WIDTH), lambda i: (0, 0)),
                pl.BlockSpec((1, WIDTH), lambda i: (0, 0)),
                pl.BlockSpec((_PRE_BLK, 1), lambda i: (i, 0)),
                pl.BlockSpec((1, WIDTH), lambda i: (0, 0)),
                pl.BlockSpec((_PRE_BLK, WIDTH), lambda i: (i, 0)),
            ]
        ),
        out_specs=pl.BlockSpec((_PRE_BLK, WIDTH), lambda i: (i, 0)),
        out_shape=jax.ShapeDtypeStruct((N, WIDTH), f32),
    )(*partials, wp_perm, b_post.reshape(1, WIDTH),
      node_deg.reshape(N, 1), degree_param.reshape(1, WIDTH), x_res)

    return out
